# baseline (speedup 1.0000x reference)
import functools

import jax
import jax.numpy as jnp
from jax import lax
from jax.experimental import pallas as pl
from jax.experimental.pallas import tpu as pltpu
from jax.experimental.pallas import tpu_sc as plsc

N = 10000
E = 160000
C = 256
H = 8
DH = C // H
HID = 1024

NP_ = 10240
EP = 163840
TILE_E = EP // 16
B = 32
NCHUNK = TILE_E // B
PAIRS = NCHUNK // 2
RW = 144
HC = C // 2
INV_SQRT_DH = 1.0 / (DH ** 0.5)


def _dyn_gather(x, idx):
    return lax.gather(
        x, idx[:, None],
        lax.GatherDimensionNumbers(offset_dims=(), collapsed_slice_dims=(0,),
                                   start_index_map=(0,)),
        (1,), mode=lax.GatherScatterMode.PROMISE_IN_BOUNDS)


def _ln(x, g, b):
    mu = jnp.mean(x, axis=-1, keepdims=True)
    xc = x - mu
    var = jnp.mean(xc * xc, axis=-1, keepdims=True)
    return xc * lax.rsqrt(var + 1e-5) * g + b


def _qkv_body(x_ref, g1_ref, b1_ref, wq_ref, bq_ref, wk_ref, bk_ref,
              wv_ref, bv_ref, q_ref, kv_ref):
    z = _ln(x_ref[...], g1_ref[...], b1_ref[...])
    yq = jnp.dot(z, wq_ref[...], preferred_element_type=jnp.float32) + bq_ref[...]
    q_ref[0] = yq[:, :HC]
    q_ref[1] = yq[:, HC:]
    yk = jnp.dot(z, wk_ref[...], preferred_element_type=jnp.float32) + bk_ref[...]
    kv_ref[0, :, :HC] = yk[:, :HC]
    kv_ref[1, :, :HC] = yk[:, HC:]
    yv = jnp.dot(z, wv_ref[...], preferred_element_type=jnp.float32) + bv_ref[...]
    kv_ref[0, :, HC:] = yv[:, :HC]
    kv_ref[1, :, HC:] = yv[:, HC:]


def _qkv_call(x_pad, g1, b1, wq, bq, wk, bk, wv, bv):
    blk = 1024
    grid = NP_ // blk
    mat = pl.BlockSpec((C, C), lambda i: (0, 0))
    vec = pl.BlockSpec((1, C), lambda i: (0, 0))
    return pl.pallas_call(
        _qkv_body,
        grid=(grid,),
        in_specs=[pl.BlockSpec((blk, C), lambda i: (i, 0)),
                  vec, vec, mat, vec, mat, vec, mat, vec],
        out_specs=[pl.BlockSpec((2, blk, HC), lambda i: (0, i, 0)),
                   pl.BlockSpec((2, blk, C), lambda i: (0, i, 0))],
        out_shape=[jax.ShapeDtypeStruct((2, NP_, HC), jnp.float32),
                   jax.ShapeDtypeStruct((2, NP_, C), jnp.float32)],
    )(x_pad, g1, b1, wq, bq, wk, bk, wv, bv)


def _sc_body(qf, kvf, edge2, biasf3, aggf,
             idx0, idx1, adjd0, adjd1, adjs0, adjs1, dstb0, dstb1,
             qd0, qd1, kv0, kv1, bb0, bb1, ub0, ub1, pbuf, aggsh,
             semi0, semi1, semg0, semg1, semu0, semu1):
    c = lax.axis_index("c")
    s = lax.axis_index("s")
    c_off = c * NP_
    iot = lax.iota(jnp.int32, 16)
    zv = jnp.zeros((16,), jnp.float32)

    @pl.loop(0, B)
    def _zero(r):
        for j in range(RW // 16):
            ub0[r, pl.ds(16 * j, 16)] = zv
            ub1[r, pl.ds(16 * j, 16)] = zv

    rows_per_tile = NP_ // 16
    for z in range(rows_per_tile // B):
        pltpu.sync_copy(ub0, aggsh.at[pl.ds(s * rows_per_tile + z * B, B)])
    plsc.subcore_barrier()

    def fire_idx(g, idxb, semi):
        pltpu.async_copy(edge2.at[:, pl.ds(s * TILE_E + g * B, B)], idxb, semi)

    def wait_idx(idxb, semi):
        pltpu.make_async_copy(edge2.at[:, pl.ds(0, B)], idxb, semi).wait()

    def adjust(idxb, adjd, adjs, dstb):
        for i in range(B // 16):
            sl = pl.ds(16 * i, 16)
            d = idxb[0, sl]
            dstb[sl] = d
            adjd[sl] = d + c_off
            adjs[sl] = idxb[1, sl] + c_off

    def fire_gathers(g, adjd, adjs, qd, kv, bb, semg):
        pltpu.async_copy(qf.at[adjd], qd, semg)
        pltpu.async_copy(kvf.at[adjs], kv, semg)
        pltpu.async_copy(biasf3.at[c, :, pl.ds(s * TILE_E + g * B, B)], bb, semg)

    def wait_gathers(adjd, adjs, qd, kv, bb, semg):
        pltpu.make_async_copy(qf.at[adjd], qd, semg).wait()
        pltpu.make_async_copy(kvf.at[adjs], kv, semg).wait()
        pltpu.make_async_copy(biasf3.at[c, :, pl.ds(0, B)], bb, semg).wait()

    def fire_scatter(ub, dstb, semu):
        pltpu.async_copy(ub, aggsh.at[dstb], semu, add=True)

    def wait_scatter(ub, dstb, semu):
        pltpu.make_async_copy(ub, aggsh.at[dstb], semu).wait()

    def compute(qd, kv, bb, ub):
        @plsc.parallel_loop(0, B, unroll=8)
        def _pp(e):
            for h in range(4):
                p = (qd[e, pl.ds(32 * h, 16)] * kv[e, pl.ds(32 * h, 16)]
                     + qd[e, pl.ds(32 * h + 16, 16)]
                     * kv[e, pl.ds(32 * h + 16, 16)])
                pbuf[pl.ds((h * B + e) * 17, 16)] = p
        exs = {}
        for h in range(4):
            for gi in range(B // 16):
                vs_ = [plsc.load_gather(pbuf,
                                        [(h * B + gi * 16 + iot) * 17 + l])
                       for l in range(16)]
                while len(vs_) > 1:
                    vs_ = [vs_[i] + vs_[i + 1]
                           for i in range(0, len(vs_), 2)]
                ex = jnp.exp(vs_[0] * INV_SQRT_DH + bb[h, pl.ds(gi * 16, 16)])
                plsc.store_scatter(ub, [gi * 16 + iot,
                                        jnp.full((16,), HC + h, jnp.int32)],
                                   ex)
                exs[(h, gi)] = ex
        @plsc.parallel_loop(0, 16, unroll=4)
        def _up(e):
            eidx = jnp.full((16,), e, jnp.int32)
            for gi in range(B // 16):
                r = gi * 16 + e
                for h in range(4):
                    spl = _dyn_gather(exs[(h, gi)], eidx)
                    ub[r, pl.ds(32 * h, 16)] = (kv[r, pl.ds(HC + 32 * h, 16)]
                                                * spl)
                    ub[r, pl.ds(32 * h + 16, 16)] = (
                        kv[r, pl.ds(HC + 32 * h + 16, 16)] * spl)

    fire_idx(0, idx0, semi0)
    wait_idx(idx0, semi0)
    adjust(idx0, adjd0, adjs0, dstb0)
    fire_gathers(0, adjd0, adjs0, qd0, kv0, bb0, semg0)
    fire_idx(1, idx1, semi1)

    @pl.loop(0, PAIRS - 1)
    def _pair(g2):
        g = 2 * g2
        wait_gathers(adjd0, adjs0, qd0, kv0, bb0, semg0)
        fire_idx(g + 2, idx0, semi0)

        @pl.when(g2 > 0)
        def _():
            wait_scatter(ub1, dstb1, semu1)

        wait_idx(idx1, semi1)
        adjust(idx1, adjd1, adjs1, dstb1)
        fire_gathers(g + 1, adjd1, adjs1, qd1, kv1, bb1, semg1)
        compute(qd0, kv0, bb0, ub0)
        fire_scatter(ub0, dstb0, semu0)
        wait_gathers(adjd1, adjs1, qd1, kv1, bb1, semg1)
        fire_idx(g + 3, idx1, semi1)
        wait_scatter(ub0, dstb0, semu0)
        wait_idx(idx0, semi0)
        adjust(idx0, adjd0, adjs0, dstb0)
        fire_gathers(g + 2, adjd0, adjs0, qd0, kv0, bb0, semg0)
        compute(qd1, kv1, bb1, ub1)
        fire_scatter(ub1, dstb1, semu1)

    wait_gathers(adjd0, adjs0, qd0, kv0, bb0, semg0)
    wait_scatter(ub1, dstb1, semu1)
    wait_idx(idx1, semi1)
    adjust(idx1, adjd1, adjs1, dstb1)
    fire_gathers(NCHUNK - 1, adjd1, adjs1, qd1, kv1, bb1, semg1)
    compute(qd0, kv0, bb0, ub0)
    fire_scatter(ub0, dstb0, semu0)

    wait_gathers(adjd1, adjs1, qd1, kv1, bb1, semg1)
    wait_scatter(ub0, dstb0, semu0)
    compute(qd1, kv1, bb1, ub1)
    fire_scatter(ub1, dstb1, semu1)
    wait_scatter(ub1, dstb1, semu1)

    plsc.subcore_barrier()
    pltpu.sync_copy(aggsh.at[pl.ds(s * rows_per_tile, rows_per_tile)],
                    aggf.at[pl.ds(c_off + s * rows_per_tile, rows_per_tile)])


_sc_kernel = functools.partial(
    pl.kernel,
    out_type=jax.ShapeDtypeStruct((2 * NP_, RW), jnp.float32),
    mesh=plsc.VectorSubcoreMesh(core_axis_name="c", subcore_axis_name="s"),
    scratch_types=[
        pltpu.VMEM((2, B), jnp.int32),
        pltpu.VMEM((2, B), jnp.int32),
        pltpu.VMEM((B,), jnp.int32),
        pltpu.VMEM((B,), jnp.int32),
        pltpu.VMEM((B,), jnp.int32),
        pltpu.VMEM((B,), jnp.int32),
        pltpu.VMEM((B,), jnp.int32),
        pltpu.VMEM((B,), jnp.int32),
        pltpu.VMEM((B, HC), jnp.float32),
        pltpu.VMEM((B, HC), jnp.float32),
        pltpu.VMEM((B, C), jnp.float32),
        pltpu.VMEM((B, C), jnp.float32),
        pltpu.VMEM((4, B), jnp.float32),
        pltpu.VMEM((4, B), jnp.float32),
        pltpu.VMEM((B, RW), jnp.float32),
        pltpu.VMEM((B, RW), jnp.float32),
        pltpu.VMEM((128 * 17,), jnp.float32),
        pltpu.VMEM_SHARED((NP_, RW), jnp.float32),
        pltpu.SemaphoreType.DMA,
        pltpu.SemaphoreType.DMA,
        pltpu.SemaphoreType.DMA,
        pltpu.SemaphoreType.DMA,
        pltpu.SemaphoreType.DMA,
        pltpu.SemaphoreType.DMA,
    ],
    compiler_params=pltpu.CompilerParams(needs_layout_passes=False,
                                         use_tc_tiling_on_sc=False),
)(_sc_body)


def _fin_body(agg_ref, x_ref, g2_ref, b2_ref, wo_ref, bo_ref,
              w1_ref, bw1_ref, w2_ref, bw2_ref, out_ref):
    parts = []
    for cc in range(2):
        a = agg_ref[cc]
        d = a[:, HC:HC + 4] + 1e-9
        for h in range(4):
            parts.append(a[:, 32 * h:32 * h + 32] / d[:, h:h + 1])
    aggn = jnp.concatenate(parts, axis=1)
    x1 = (x_ref[...]
          + jnp.dot(aggn, wo_ref[...], preferred_element_type=jnp.float32)
          + bo_ref[...])
    z2 = _ln(x1, g2_ref[...], b2_ref[...])
    h1 = jnp.dot(z2, w1_ref[...], preferred_element_type=jnp.float32) + bw1_ref[...]
    h1 = h1 * jax.nn.sigmoid(h1)
    out_ref[...] = (x1
                    + jnp.dot(h1, w2_ref[...], preferred_element_type=jnp.float32)
                    + bw2_ref[...])


def _fin_call(agg3, x, g2, b2, wo, bo, w1, bw1, w2, bw2):
    blk = 1000
    grid = N // blk
    vec = pl.BlockSpec((1, C), lambda i: (0, 0))
    return pl.pallas_call(
        _fin_body,
        grid=(grid,),
        in_specs=[pl.BlockSpec((2, blk, RW), lambda i: (0, i, 0)),
                  pl.BlockSpec((blk, C), lambda i: (i, 0)),
                  vec, vec,
                  pl.BlockSpec((C, C), lambda i: (0, 0)), vec,
                  pl.BlockSpec((C, HID), lambda i: (0, 0)),
                  pl.BlockSpec((1, HID), lambda i: (0, 0)),
                  pl.BlockSpec((HID, C), lambda i: (0, 0)), vec],
        out_specs=pl.BlockSpec((blk, C), lambda i: (i, 0)),
        out_shape=jax.ShapeDtypeStruct((N, C), jnp.float32),
    )(agg3, x, g2, b2, wo, bo, w1, bw1, w2, bw2)


def kernel(x, edge_index, att_bias, g1, b1, g2, b2, Wq, bq, Wk, bk, Wv, bv,
           Wo, bo, W1, bw1, W2, bw2):
    f32 = jnp.float32
    x_pad = jnp.zeros((NP_, C), f32).at[:N].set(x)
    g1r, b1r = g1.reshape(1, C), b1.reshape(1, C)
    g2r, b2r = g2.reshape(1, C), b2.reshape(1, C)
    bqr, bkr, bvr, bor = (t.reshape(1, C) for t in (bq, bk, bv, bo))
    bw1r, bw2r = bw1.reshape(1, HID), bw2.reshape(1, C)

    q2, kv2 = _qkv_call(x_pad, g1r, b1r, Wq, bqr, Wk, bkr, Wv, bvr)
    qf = q2.reshape(2 * NP_, HC)
    kvf = kv2.reshape(2 * NP_, C)

    pad_e = EP - E
    dstp = jnp.concatenate([edge_index[1], jnp.full((pad_e,), N, jnp.int32)])
    srcp = jnp.concatenate([edge_index[0], jnp.full((pad_e,), N, jnp.int32)])
    edge2 = jnp.stack([dstp, srcp])
    biasf3 = jnp.zeros((2, 4, EP), f32).at[:, :, :E].set(
        att_bias.T.reshape(2, 4, E))

    aggf = _sc_kernel(qf, kvf, edge2, biasf3)
    agg3 = aggf.reshape(2, NP_, RW)

    return _fin_call(agg3, x, g2r, b2r, Wo, bor, W1, bw1r, W2, bw2r)


# speedup vs baseline: 1.0428x; 1.0428x over previous
import functools

import jax
import jax.numpy as jnp
from jax import lax
from jax.experimental import pallas as pl
from jax.experimental.pallas import tpu as pltpu
from jax.experimental.pallas import tpu_sc as plsc

N = 10000
E = 160000
C = 256
H = 8
DH = C // H
HID = 1024

NP_ = 10240
EP = 163840
TILE_E = EP // 16
B = 32
NCHUNK = TILE_E // B
PAIRS = NCHUNK // 2
RW = 144
HC = C // 2
INV_SQRT_DH = 1.0 / (DH ** 0.5)


def _dyn_gather(x, idx):
    return lax.gather(
        x, idx[:, None],
        lax.GatherDimensionNumbers(offset_dims=(), collapsed_slice_dims=(0,),
                                   start_index_map=(0,)),
        (1,), mode=lax.GatherScatterMode.PROMISE_IN_BOUNDS)


def _ln(x, g, b):
    mu = jnp.mean(x, axis=-1, keepdims=True)
    xc = x - mu
    var = jnp.mean(xc * xc, axis=-1, keepdims=True)
    return xc * lax.rsqrt(var + 1e-5) * g + b


def _qkv_body(x_ref, g1_ref, b1_ref, wq_ref, bq_ref, wk_ref, bk_ref,
              wv_ref, bv_ref, q_ref, kv_ref):
    z = _ln(x_ref[...], g1_ref[...], b1_ref[...])
    yq = jnp.dot(z, wq_ref[...], preferred_element_type=jnp.float32) + bq_ref[...]
    q_ref[0] = yq[:, :HC]
    q_ref[1] = yq[:, HC:]
    yk = jnp.dot(z, wk_ref[...], preferred_element_type=jnp.float32) + bk_ref[...]
    kv_ref[0, :, :HC] = yk[:, :HC]
    kv_ref[1, :, :HC] = yk[:, HC:]
    yv = jnp.dot(z, wv_ref[...], preferred_element_type=jnp.float32) + bv_ref[...]
    kv_ref[0, :, HC:] = yv[:, :HC]
    kv_ref[1, :, HC:] = yv[:, HC:]


def _qkv_call(x_pad, g1, b1, wq, bq, wk, bk, wv, bv):
    blk = 1024
    grid = NP_ // blk
    mat = pl.BlockSpec((C, C), lambda i: (0, 0))
    vec = pl.BlockSpec((1, C), lambda i: (0, 0))
    return pl.pallas_call(
        _qkv_body,
        grid=(grid,),
        in_specs=[pl.BlockSpec((blk, C), lambda i: (i, 0)),
                  vec, vec, mat, vec, mat, vec, mat, vec],
        out_specs=[pl.BlockSpec((2, blk, HC), lambda i: (0, i, 0)),
                   pl.BlockSpec((2, blk, C), lambda i: (0, i, 0))],
        out_shape=[jax.ShapeDtypeStruct((2, NP_, HC), jnp.float32),
                   jax.ShapeDtypeStruct((2, NP_, C), jnp.float32)],
    )(x_pad, g1, b1, wq, bq, wk, bk, wv, bv)


def _sc_body(qf, kvf, edge2, biasf3, aggf,
             idx0, idx1, adjd0, adjd1, adjs0, adjs1, dstb0, dstb1,
             qd0, qd1, kv0, kv1, bb0, bb1, ub0, ub1, pbuf, xbuf, aggsh,
             semi0, semi1, semg0, semg1, semu0, semu1):
    c = lax.axis_index("c")
    s = lax.axis_index("s")
    c_off = c * NP_
    iot = lax.iota(jnp.int32, 16)
    zv = jnp.zeros((16,), jnp.float32)

    @pl.loop(0, B)
    def _zero(r):
        for j in range(RW // 16):
            ub0[r, pl.ds(16 * j, 16)] = zv
            ub1[r, pl.ds(16 * j, 16)] = zv

    xbuf[pl.ds(64, 16)] = zv

    rows_per_tile = NP_ // 16
    for z in range(rows_per_tile // B):
        pltpu.sync_copy(ub0, aggsh.at[pl.ds(s * rows_per_tile + z * B, B)])
    plsc.subcore_barrier()

    def fire_idx(g, idxb, semi):
        pltpu.async_copy(edge2.at[:, pl.ds(s * TILE_E + g * B, B)], idxb, semi)

    def wait_idx(idxb, semi):
        pltpu.make_async_copy(edge2.at[:, pl.ds(0, B)], idxb, semi).wait()

    def adjust(idxb, adjd, adjs, dstb):
        for i in range(B // 16):
            sl = pl.ds(16 * i, 16)
            d = idxb[0, sl]
            dstb[sl] = d
            adjd[sl] = d + c_off
            adjs[sl] = idxb[1, sl] + c_off

    def fire_gathers(g, adjd, adjs, qd, kv, bb, semg):
        pltpu.async_copy(qf.at[adjd], qd, semg)
        pltpu.async_copy(kvf.at[adjs], kv, semg)
        pltpu.async_copy(biasf3.at[c, :, pl.ds(s * TILE_E + g * B, B)], bb, semg)

    def wait_gathers(adjd, adjs, qd, kv, bb, semg):
        pltpu.make_async_copy(qf.at[adjd], qd, semg).wait()
        pltpu.make_async_copy(kvf.at[adjs], kv, semg).wait()
        pltpu.make_async_copy(biasf3.at[c, :, pl.ds(0, B)], bb, semg).wait()

    def fire_scatter(ub, dstb, semu):
        pltpu.async_copy(ub, aggsh.at[dstb], semu, add=True)

    def wait_scatter(ub, dstb, semu):
        pltpu.make_async_copy(ub, aggsh.at[dstb], semu).wait()

    def compute(qd, kv, bb, ub):
        @pl.loop(0, B // 16)
        def _group(gi):
            eb = 16 * gi
            @plsc.parallel_loop(0, 16, unroll=4)
            def _pp(e):
                r = eb + e
                for h in range(4):
                    p = (qd[r, pl.ds(32 * h, 16)] * kv[r, pl.ds(32 * h, 16)]
                         + qd[r, pl.ds(32 * h + 16, 16)]
                         * kv[r, pl.ds(32 * h + 16, 16)])
                    pbuf[pl.ds((h * 16 + e) * 17, 16)] = p
            cols = [[plsc.load_gather(pbuf, [(h * 16 + iot) * 17 + l])
                     for h in range(4)] for l in range(16)]
            exs = []
            for h in range(4):
                vs_ = [cols[l][h] for l in range(16)]
                while len(vs_) > 1:
                    vs_ = [vs_[i] + vs_[i + 1] for i in range(0, len(vs_), 2)]
                ex = jnp.exp(vs_[0] * INV_SQRT_DH + bb[h, pl.ds(eb, 16)])
                xbuf[pl.ds(h * 16, 16)] = ex
                exs.append(ex)
            @plsc.parallel_loop(0, 16, unroll=4)
            def _up(e):
                r = eb + e
                eidx = jnp.full((16,), e, jnp.int32)
                for h in range(4):
                    spl = _dyn_gather(exs[h], eidx)
                    ub[r, pl.ds(32 * h, 16)] = (kv[r, pl.ds(HC + 32 * h, 16)]
                                                * spl)
                    ub[r, pl.ds(32 * h + 16, 16)] = (
                        kv[r, pl.ds(HC + 32 * h + 16, 16)] * spl)
                ext = plsc.load_gather(
                    xbuf, [jnp.where(iot < 4, iot * 16 + e, 64 + iot)])
                ub[r, pl.ds(HC, 16)] = ext

    fire_idx(0, idx0, semi0)
    wait_idx(idx0, semi0)
    adjust(idx0, adjd0, adjs0, dstb0)
    fire_gathers(0, adjd0, adjs0, qd0, kv0, bb0, semg0)
    fire_idx(1, idx1, semi1)

    @pl.loop(0, PAIRS - 1)
    def _pair(g2):
        g = 2 * g2
        wait_gathers(adjd0, adjs0, qd0, kv0, bb0, semg0)
        fire_idx(g + 2, idx0, semi0)

        @pl.when(g2 > 0)
        def _():
            wait_scatter(ub1, dstb1, semu1)

        wait_idx(idx1, semi1)
        adjust(idx1, adjd1, adjs1, dstb1)
        fire_gathers(g + 1, adjd1, adjs1, qd1, kv1, bb1, semg1)
        compute(qd0, kv0, bb0, ub0)
        fire_scatter(ub0, dstb0, semu0)
        wait_gathers(adjd1, adjs1, qd1, kv1, bb1, semg1)
        fire_idx(g + 3, idx1, semi1)
        wait_scatter(ub0, dstb0, semu0)
        wait_idx(idx0, semi0)
        adjust(idx0, adjd0, adjs0, dstb0)
        fire_gathers(g + 2, adjd0, adjs0, qd0, kv0, bb0, semg0)
        compute(qd1, kv1, bb1, ub1)
        fire_scatter(ub1, dstb1, semu1)

    wait_gathers(adjd0, adjs0, qd0, kv0, bb0, semg0)
    wait_scatter(ub1, dstb1, semu1)
    wait_idx(idx1, semi1)
    adjust(idx1, adjd1, adjs1, dstb1)
    fire_gathers(NCHUNK - 1, adjd1, adjs1, qd1, kv1, bb1, semg1)
    compute(qd0, kv0, bb0, ub0)
    fire_scatter(ub0, dstb0, semu0)

    wait_gathers(adjd1, adjs1, qd1, kv1, bb1, semg1)
    wait_scatter(ub0, dstb0, semu0)
    compute(qd1, kv1, bb1, ub1)
    fire_scatter(ub1, dstb1, semu1)
    wait_scatter(ub1, dstb1, semu1)

    plsc.subcore_barrier()
    pltpu.sync_copy(aggsh.at[pl.ds(s * rows_per_tile, rows_per_tile)],
                    aggf.at[pl.ds(c_off + s * rows_per_tile, rows_per_tile)])


_sc_kernel = functools.partial(
    pl.kernel,
    out_type=jax.ShapeDtypeStruct((2 * NP_, RW), jnp.float32),
    mesh=plsc.VectorSubcoreMesh(core_axis_name="c", subcore_axis_name="s"),
    scratch_types=[
        pltpu.VMEM((2, B), jnp.int32),
        pltpu.VMEM((2, B), jnp.int32),
        pltpu.VMEM((B,), jnp.int32),
        pltpu.VMEM((B,), jnp.int32),
        pltpu.VMEM((B,), jnp.int32),
        pltpu.VMEM((B,), jnp.int32),
        pltpu.VMEM((B,), jnp.int32),
        pltpu.VMEM((B,), jnp.int32),
        pltpu.VMEM((B, HC), jnp.float32),
        pltpu.VMEM((B, HC), jnp.float32),
        pltpu.VMEM((B, C), jnp.float32),
        pltpu.VMEM((B, C), jnp.float32),
        pltpu.VMEM((4, B), jnp.float32),
        pltpu.VMEM((4, B), jnp.float32),
        pltpu.VMEM((B, RW), jnp.float32),
        pltpu.VMEM((B, RW), jnp.float32),
        pltpu.VMEM((64 * 17,), jnp.float32),
        pltpu.VMEM((80,), jnp.float32),
        pltpu.VMEM_SHARED((NP_, RW), jnp.float32),
        pltpu.SemaphoreType.DMA,
        pltpu.SemaphoreType.DMA,
        pltpu.SemaphoreType.DMA,
        pltpu.SemaphoreType.DMA,
        pltpu.SemaphoreType.DMA,
        pltpu.SemaphoreType.DMA,
    ],
    compiler_params=pltpu.CompilerParams(needs_layout_passes=False,
                                         use_tc_tiling_on_sc=False),
)(_sc_body)


def _fin_body(agg_ref, x_ref, g2_ref, b2_ref, wo_ref, bo_ref,
              w1_ref, bw1_ref, w2_ref, bw2_ref, out_ref):
    parts = []
    for cc in range(2):
        a = agg_ref[cc]
        d = a[:, HC:HC + 4] + 1e-9
        for h in range(4):
            parts.append(a[:, 32 * h:32 * h + 32] / d[:, h:h + 1])
    aggn = jnp.concatenate(parts, axis=1)
    x1 = (x_ref[...]
          + jnp.dot(aggn, wo_ref[...], preferred_element_type=jnp.float32)
          + bo_ref[...])
    z2 = _ln(x1, g2_ref[...], b2_ref[...])
    h1 = jnp.dot(z2, w1_ref[...], preferred_element_type=jnp.float32) + bw1_ref[...]
    h1 = h1 * jax.nn.sigmoid(h1)
    out_ref[...] = (x1
                    + jnp.dot(h1, w2_ref[...], preferred_element_type=jnp.float32)
                    + bw2_ref[...])


def _fin_call(agg3, x, g2, b2, wo, bo, w1, bw1, w2, bw2):
    blk = 1000
    grid = N // blk
    vec = pl.BlockSpec((1, C), lambda i: (0, 0))
    return pl.pallas_call(
        _fin_body,
        grid=(grid,),
        in_specs=[pl.BlockSpec((2, blk, RW), lambda i: (0, i, 0)),
                  pl.BlockSpec((blk, C), lambda i: (i, 0)),
                  vec, vec,
                  pl.BlockSpec((C, C), lambda i: (0, 0)), vec,
                  pl.BlockSpec((C, HID), lambda i: (0, 0)),
                  pl.BlockSpec((1, HID), lambda i: (0, 0)),
                  pl.BlockSpec((HID, C), lambda i: (0, 0)), vec],
        out_specs=pl.BlockSpec((blk, C), lambda i: (i, 0)),
        out_shape=jax.ShapeDtypeStruct((N, C), jnp.float32),
    )(agg3, x, g2, b2, wo, bo, w1, bw1, w2, bw2)


def kernel(x, edge_index, att_bias, g1, b1, g2, b2, Wq, bq, Wk, bk, Wv, bv,
           Wo, bo, W1, bw1, W2, bw2):
    f32 = jnp.float32
    x_pad = jnp.zeros((NP_, C), f32).at[:N].set(x)
    g1r, b1r = g1.reshape(1, C), b1.reshape(1, C)
    g2r, b2r = g2.reshape(1, C), b2.reshape(1, C)
    bqr, bkr, bvr, bor = (t.reshape(1, C) for t in (bq, bk, bv, bo))
    bw1r, bw2r = bw1.reshape(1, HID), bw2.reshape(1, C)

    q2, kv2 = _qkv_call(x_pad, g1r, b1r, Wq, bqr, Wk, bkr, Wv, bvr)
    qf = q2.reshape(2 * NP_, HC)
    kvf = kv2.reshape(2 * NP_, C)

    pad_e = EP - E
    dstp = jnp.concatenate([edge_index[1], jnp.full((pad_e,), N, jnp.int32)])
    srcp = jnp.concatenate([edge_index[0], jnp.full((pad_e,), N, jnp.int32)])
    edge2 = jnp.stack([dstp, srcp])
    biasf3 = jnp.zeros((2, 4, EP), f32).at[:, :, :E].set(
        att_bias.T.reshape(2, 4, E))

    aggf = _sc_kernel(qf, kvf, edge2, biasf3)
    agg3 = aggf.reshape(2, NP_, RW)

    return _fin_call(agg3, x, g2r, b2r, Wo, bor, W1, bw1r, W2, bw2r)


# speedup vs baseline: 1.0436x; 1.0008x over previous
import functools

import jax
import jax.numpy as jnp
from jax import lax
from jax.experimental import pallas as pl
from jax.experimental.pallas import tpu as pltpu
from jax.experimental.pallas import tpu_sc as plsc

N = 10000
E = 160000
C = 256
H = 8
DH = C // H
HID = 1024

NP_ = 10240
EP = 163840
TILE_E = EP // 16
B = 32
NCHUNK = TILE_E // B
PAIRS = NCHUNK // 2
RW = 144
HC = C // 2
INV_SQRT_DH = 1.0 / (DH ** 0.5)


def _dyn_gather(x, idx):
    return lax.gather(
        x, idx[:, None],
        lax.GatherDimensionNumbers(offset_dims=(), collapsed_slice_dims=(0,),
                                   start_index_map=(0,)),
        (1,), mode=lax.GatherScatterMode.PROMISE_IN_BOUNDS)


def _ln(x, g, b):
    mu = jnp.mean(x, axis=-1, keepdims=True)
    xc = x - mu
    var = jnp.mean(xc * xc, axis=-1, keepdims=True)
    return xc * lax.rsqrt(var + 1e-5) * g + b


def _qkv_body(x_ref, g1_ref, b1_ref, wq_ref, bq_ref, wk_ref, bk_ref,
              wv_ref, bv_ref, q_ref, kv_ref):
    z = _ln(x_ref[...], g1_ref[...], b1_ref[...])
    yq = jnp.dot(z, wq_ref[...], preferred_element_type=jnp.float32) + bq_ref[...]
    q_ref[0] = yq[:, :HC]
    q_ref[1] = yq[:, HC:]
    yk = jnp.dot(z, wk_ref[...], preferred_element_type=jnp.float32) + bk_ref[...]
    kv_ref[0, :, :HC] = yk[:, :HC]
    kv_ref[1, :, :HC] = yk[:, HC:]
    yv = jnp.dot(z, wv_ref[...], preferred_element_type=jnp.float32) + bv_ref[...]
    kv_ref[0, :, HC:] = yv[:, :HC]
    kv_ref[1, :, HC:] = yv[:, HC:]


def _qkv_call(x_pad, g1, b1, wq, bq, wk, bk, wv, bv):
    blk = 1024
    grid = NP_ // blk
    mat = pl.BlockSpec((C, C), lambda i: (0, 0))
    vec = pl.BlockSpec((1, C), lambda i: (0, 0))
    return pl.pallas_call(
        _qkv_body,
        grid=(grid,),
        in_specs=[pl.BlockSpec((blk, C), lambda i: (i, 0)),
                  vec, vec, mat, vec, mat, vec, mat, vec],
        out_specs=[pl.BlockSpec((2, blk, HC), lambda i: (0, i, 0)),
                   pl.BlockSpec((2, blk, C), lambda i: (0, i, 0))],
        out_shape=[jax.ShapeDtypeStruct((2, NP_, HC), jnp.float32),
                   jax.ShapeDtypeStruct((2, NP_, C), jnp.float32)],
    )(x_pad, g1, b1, wq, bq, wk, bk, wv, bv)


def _sc_body(qf, kvf, edge2, biasf3, aggf,
             idx0, idx1, adjd0, adjd1, adjs0, adjs1, dstb0, dstb1,
             qd0, qd1, kv0, kv1, bb0, bb1, ub0, ub1, pbuf, xbuf, aggsh,
             semi0, semi1, semg0, semg1, semu0, semu1):
    c = lax.axis_index("c")
    s = lax.axis_index("s")
    c_off = c * NP_
    iot = lax.iota(jnp.int32, 16)
    zv = jnp.zeros((16,), jnp.float32)

    @pl.loop(0, B)
    def _zero(r):
        for j in range(RW // 16):
            ub0[r, pl.ds(16 * j, 16)] = zv
            ub1[r, pl.ds(16 * j, 16)] = zv

    xbuf[pl.ds(64, 16)] = zv

    rows_per_tile = NP_ // 16
    for z in range(rows_per_tile // B):
        pltpu.sync_copy(ub0, aggsh.at[pl.ds(s * rows_per_tile + z * B, B)])
    plsc.subcore_barrier()

    def fire_idx(g, idxb, semi):
        pltpu.async_copy(edge2.at[:, pl.ds(s * TILE_E + g * B, B)], idxb, semi)

    def wait_idx(idxb, semi):
        pltpu.make_async_copy(edge2.at[:, pl.ds(0, B)], idxb, semi).wait()

    def adjust(idxb, adjd, adjs, dstb):
        for i in range(B // 16):
            sl = pl.ds(16 * i, 16)
            d = idxb[0, sl]
            dstb[sl] = d
            adjd[sl] = d + c_off
            adjs[sl] = idxb[1, sl] + c_off

    def fire_gathers(g, adjd, adjs, qd, kv, bb, semg):
        pltpu.async_copy(qf.at[adjd], qd, semg)
        pltpu.async_copy(kvf.at[adjs], kv, semg)
        pltpu.async_copy(biasf3.at[c, :, pl.ds(s * TILE_E + g * B, B)], bb, semg)

    def wait_gathers(adjd, adjs, qd, kv, bb, semg):
        pltpu.make_async_copy(qf.at[adjd], qd, semg).wait()
        pltpu.make_async_copy(kvf.at[adjs], kv, semg).wait()
        pltpu.make_async_copy(biasf3.at[c, :, pl.ds(0, B)], bb, semg).wait()

    def fire_scatter(ub, dstb, semu):
        pltpu.async_copy(ub, aggsh.at[dstb], semu, add=True)

    def wait_scatter(ub, dstb, semu):
        pltpu.make_async_copy(ub, aggsh.at[dstb], semu).wait()

    def compute(qd, kv, bb, ub):
        @pl.loop(0, B // 16)
        def _group(gi):
            eb = 16 * gi
            @plsc.parallel_loop(0, 16, unroll=8)
            def _pp(e):
                r = eb + e
                for h in range(4):
                    p = (qd[r, pl.ds(32 * h, 16)] * kv[r, pl.ds(32 * h, 16)]
                         + qd[r, pl.ds(32 * h + 16, 16)]
                         * kv[r, pl.ds(32 * h + 16, 16)])
                    pbuf[pl.ds((h * 16 + e) * 17, 16)] = p
            cols = [[plsc.load_gather(pbuf, [(h * 16 + iot) * 17 + l])
                     for h in range(4)] for l in range(16)]
            exs = []
            for h in range(4):
                vs_ = [cols[l][h] for l in range(16)]
                while len(vs_) > 1:
                    vs_ = [vs_[i] + vs_[i + 1] for i in range(0, len(vs_), 2)]
                ex = jnp.exp(vs_[0] * INV_SQRT_DH + bb[h, pl.ds(eb, 16)])
                xbuf[pl.ds(h * 16, 16)] = ex
                exs.append(ex)
            @plsc.parallel_loop(0, 16, unroll=8)
            def _up(e):
                r = eb + e
                eidx = jnp.full((16,), e, jnp.int32)
                for h in range(4):
                    spl = _dyn_gather(exs[h], eidx)
                    ub[r, pl.ds(32 * h, 16)] = (kv[r, pl.ds(HC + 32 * h, 16)]
                                                * spl)
                    ub[r, pl.ds(32 * h + 16, 16)] = (
                        kv[r, pl.ds(HC + 32 * h + 16, 16)] * spl)
                ext = plsc.load_gather(
                    xbuf, [jnp.where(iot < 4, iot * 16 + e, 64 + iot)])
                ub[r, pl.ds(HC, 16)] = ext

    fire_idx(0, idx0, semi0)
    wait_idx(idx0, semi0)
    adjust(idx0, adjd0, adjs0, dstb0)
    fire_gathers(0, adjd0, adjs0, qd0, kv0, bb0, semg0)
    fire_idx(1, idx1, semi1)

    @pl.loop(0, PAIRS - 1)
    def _pair(g2):
        g = 2 * g2
        wait_gathers(adjd0, adjs0, qd0, kv0, bb0, semg0)
        fire_idx(g + 2, idx0, semi0)

        @pl.when(g2 > 0)
        def _():
            wait_scatter(ub1, dstb1, semu1)

        wait_idx(idx1, semi1)
        adjust(idx1, adjd1, adjs1, dstb1)
        fire_gathers(g + 1, adjd1, adjs1, qd1, kv1, bb1, semg1)
        compute(qd0, kv0, bb0, ub0)
        fire_scatter(ub0, dstb0, semu0)
        wait_gathers(adjd1, adjs1, qd1, kv1, bb1, semg1)
        fire_idx(g + 3, idx1, semi1)
        wait_scatter(ub0, dstb0, semu0)
        wait_idx(idx0, semi0)
        adjust(idx0, adjd0, adjs0, dstb0)
        fire_gathers(g + 2, adjd0, adjs0, qd0, kv0, bb0, semg0)
        compute(qd1, kv1, bb1, ub1)
        fire_scatter(ub1, dstb1, semu1)

    wait_gathers(adjd0, adjs0, qd0, kv0, bb0, semg0)
    wait_scatter(ub1, dstb1, semu1)
    wait_idx(idx1, semi1)
    adjust(idx1, adjd1, adjs1, dstb1)
    fire_gathers(NCHUNK - 1, adjd1, adjs1, qd1, kv1, bb1, semg1)
    compute(qd0, kv0, bb0, ub0)
    fire_scatter(ub0, dstb0, semu0)

    wait_gathers(adjd1, adjs1, qd1, kv1, bb1, semg1)
    wait_scatter(ub0, dstb0, semu0)
    compute(qd1, kv1, bb1, ub1)
    fire_scatter(ub1, dstb1, semu1)
    wait_scatter(ub1, dstb1, semu1)

    plsc.subcore_barrier()
    pltpu.sync_copy(aggsh.at[pl.ds(s * rows_per_tile, rows_per_tile)],
                    aggf.at[pl.ds(c_off + s * rows_per_tile, rows_per_tile)])


_sc_kernel = functools.partial(
    pl.kernel,
    out_type=jax.ShapeDtypeStruct((2 * NP_, RW), jnp.float32),
    mesh=plsc.VectorSubcoreMesh(core_axis_name="c", subcore_axis_name="s"),
    scratch_types=[
        pltpu.VMEM((2, B), jnp.int32),
        pltpu.VMEM((2, B), jnp.int32),
        pltpu.VMEM((B,), jnp.int32),
        pltpu.VMEM((B,), jnp.int32),
        pltpu.VMEM((B,), jnp.int32),
        pltpu.VMEM((B,), jnp.int32),
        pltpu.VMEM((B,), jnp.int32),
        pltpu.VMEM((B,), jnp.int32),
        pltpu.VMEM((B, HC), jnp.float32),
        pltpu.VMEM((B, HC), jnp.float32),
        pltpu.VMEM((B, C), jnp.float32),
        pltpu.VMEM((B, C), jnp.float32),
        pltpu.VMEM((4, B), jnp.float32),
        pltpu.VMEM((4, B), jnp.float32),
        pltpu.VMEM((B, RW), jnp.float32),
        pltpu.VMEM((B, RW), jnp.float32),
        pltpu.VMEM((64 * 17,), jnp.float32),
        pltpu.VMEM((80,), jnp.float32),
        pltpu.VMEM_SHARED((NP_, RW), jnp.float32),
        pltpu.SemaphoreType.DMA,
        pltpu.SemaphoreType.DMA,
        pltpu.SemaphoreType.DMA,
        pltpu.SemaphoreType.DMA,
        pltpu.SemaphoreType.DMA,
        pltpu.SemaphoreType.DMA,
    ],
    compiler_params=pltpu.CompilerParams(needs_layout_passes=False,
                                         use_tc_tiling_on_sc=False),
)(_sc_body)


def _fin_body(agg_ref, x_ref, g2_ref, b2_ref, wo_ref, bo_ref,
              w1_ref, bw1_ref, w2_ref, bw2_ref, out_ref):
    parts = []
    for cc in range(2):
        a = agg_ref[cc]
        d = a[:, HC:HC + 4] + 1e-9
        for h in range(4):
            parts.append(a[:, 32 * h:32 * h + 32] / d[:, h:h + 1])
    aggn = jnp.concatenate(parts, axis=1)
    x1 = (x_ref[...]
          + jnp.dot(aggn, wo_ref[...], preferred_element_type=jnp.float32)
          + bo_ref[...])
    z2 = _ln(x1, g2_ref[...], b2_ref[...])
    h1 = jnp.dot(z2, w1_ref[...], preferred_element_type=jnp.float32) + bw1_ref[...]
    h1 = h1 * jax.nn.sigmoid(h1)
    out_ref[...] = (x1
                    + jnp.dot(h1, w2_ref[...], preferred_element_type=jnp.float32)
                    + bw2_ref[...])


def _fin_call(agg3, x, g2, b2, wo, bo, w1, bw1, w2, bw2):
    blk = 1000
    grid = N // blk
    vec = pl.BlockSpec((1, C), lambda i: (0, 0))
    return pl.pallas_call(
        _fin_body,
        grid=(grid,),
        in_specs=[pl.BlockSpec((2, blk, RW), lambda i: (0, i, 0)),
                  pl.BlockSpec((blk, C), lambda i: (i, 0)),
                  vec, vec,
                  pl.BlockSpec((C, C), lambda i: (0, 0)), vec,
                  pl.BlockSpec((C, HID), lambda i: (0, 0)),
                  pl.BlockSpec((1, HID), lambda i: (0, 0)),
                  pl.BlockSpec((HID, C), lambda i: (0, 0)), vec],
        out_specs=pl.BlockSpec((blk, C), lambda i: (i, 0)),
        out_shape=jax.ShapeDtypeStruct((N, C), jnp.float32),
    )(agg3, x, g2, b2, wo, bo, w1, bw1, w2, bw2)


def kernel(x, edge_index, att_bias, g1, b1, g2, b2, Wq, bq, Wk, bk, Wv, bv,
           Wo, bo, W1, bw1, W2, bw2):
    f32 = jnp.float32
    x_pad = jnp.zeros((NP_, C), f32).at[:N].set(x)
    g1r, b1r = g1.reshape(1, C), b1.reshape(1, C)
    g2r, b2r = g2.reshape(1, C), b2.reshape(1, C)
    bqr, bkr, bvr, bor = (t.reshape(1, C) for t in (bq, bk, bv, bo))
    bw1r, bw2r = bw1.reshape(1, HID), bw2.reshape(1, C)

    q2, kv2 = _qkv_call(x_pad, g1r, b1r, Wq, bqr, Wk, bkr, Wv, bvr)
    qf = q2.reshape(2 * NP_, HC)
    kvf = kv2.reshape(2 * NP_, C)

    pad_e = EP - E
    dstp = jnp.concatenate([edge_index[1], jnp.full((pad_e,), N, jnp.int32)])
    srcp = jnp.concatenate([edge_index[0], jnp.full((pad_e,), N, jnp.int32)])
    edge2 = jnp.stack([dstp, srcp])
    biasf3 = jnp.zeros((2, 4, EP), f32).at[:, :, :E].set(
        att_bias.T.reshape(2, 4, E))

    aggf = _sc_kernel(qf, kvf, edge2, biasf3)
    agg3 = aggf.reshape(2, NP_, RW)

    return _fin_call(agg3, x, g2r, b2r, Wo, bor, W1, bw1r, W2, bw2r)
